# baseline (speedup 1.0000x reference)
import dataclasses
import functools
import math

import jax
import jax.numpy as jnp
from jax import lax
from jax.experimental import pallas as pl
from jax.experimental.pallas import tpu as pltpu
from jax.experimental.pallas import tpu_sc as plsc

_S = 64.0
_M = 0.15
_A = _S * math.log2(math.e)


def _sc_gather(xt, target):
    c, b = xt.shape
    info = plsc.get_sparse_core_info()
    nw = info.num_cores * info.num_subcores
    b_per_w = b // nw
    n_groups = b_per_w // 16
    mesh = plsc.VectorSubcoreMesh(core_axis_name="c", subcore_axis_name="s")
    cp = pltpu.CompilerParams()
    if "needs_layout_passes" in pltpu.CompilerParams.__dataclass_fields__:
        cp = dataclasses.replace(cp, needs_layout_passes=False)

    @functools.partial(
        pl.kernel,
        mesh=mesh,
        compiler_params=cp,
        out_type=jax.ShapeDtypeStruct((b,), jnp.float32),
        scratch_types=[
            pltpu.VMEM((b_per_w,), jnp.int32),
            pltpu.VMEM((b_per_w, b), jnp.float32),
            pltpu.VMEM((b_per_w,), jnp.float32),
        ],
    )
    def gather_kernel(xt_hbm, tgt_hbm, out_hbm, idx_v, rows_v, val_v):
        wid = lax.axis_index("s") * info.num_cores + lax.axis_index("c")
        base = wid * b_per_w
        pltpu.sync_copy(tgt_hbm.at[pl.ds(base, b_per_w)], idx_v)
        pltpu.sync_copy(xt_hbm.at[idx_v], rows_v)
        lane16 = lax.iota(jnp.int32, 16)
        for g in range(n_groups):
            row_idx = lane16 + (g * 16)
            col_idx = lane16 + (base + g * 16)
            vals = plsc.load_gather(rows_v, [row_idx, col_idx])
            val_v[pl.ds(g * 16, 16)] = vals
        pltpu.sync_copy(val_v, out_hbm.at[pl.ds(base, b_per_w)])

    return gather_kernel(xt, target)


def _stream_kernel(*refs, c_total, block_r, n_streams):
    x_refs = refs[:n_streams]
    m_ref, s_ref, m_sc, s_sc = refs[n_streams:]
    k = pl.program_id(0)
    nk = pl.num_programs(0)

    @pl.when(k == 0)
    def _init():
        m_sc[...] = jnp.full_like(m_sc, -jnp.inf)
        s_sc[...] = jnp.zeros_like(s_sc)

    def process(x):
        m_old = m_sc[...]
        m_new = jnp.maximum(m_old, jnp.max(x, axis=0, keepdims=True))
        y = jnp.exp2(x * _A - m_new * _A)
        s_sc[...] = s_sc[...] * jnp.exp2(_A * (m_old - m_new)) + jnp.sum(
            y, axis=0, keepdims=True)
        m_sc[...] = m_new

    if c_total == n_streams * nk * block_r:
        for x_ref in x_refs:
            process(x_ref[...])
    else:
        @pl.when(k < nk - 1)
        def _steady():
            for x_ref in x_refs:
                process(x_ref[...])

        @pl.when(k == nk - 1)
        def _last():
            valid = c_total - n_streams * (nk - 1) * block_r
            for j, x_ref in enumerate(x_refs):
                x = x_ref[...]
                pad = (jax.lax.broadcasted_iota(jnp.int32, x.shape, 0)
                       >= valid - j * block_r)
                process(jnp.where(pad, -jnp.inf, x))

    @pl.when(k == nk - 1)
    def _emit():
        m_ref[...] = m_sc[...]
        s_ref[...] = s_sc[...]


def _epilogue_kernel(tval_ref, m_ref, s_ref, out_ref):
    m = _S * m_ref[...]
    tv = _S * tval_ref[...]
    tm = tv - _S * _M
    se = s_ref[...] - jnp.exp(tv - m) + jnp.exp(tm - m)
    logpt = tm - m - jnp.log(se)
    out_ref[...] = jnp.full_like(out_ref, -jnp.mean(logpt))


def kernel(cos_theta, target):
    b, c = cos_theta.shape
    xt = cos_theta.T
    tval = _sc_gather(xt, target.astype(jnp.int32))

    n_streams = 4
    block_r = min(1000, c)
    grid = pl.cdiv(c, n_streams * block_r)

    m_v, s_v = pl.pallas_call(
        functools.partial(_stream_kernel, c_total=c, block_r=block_r,
                          n_streams=n_streams),
        grid=(grid,),
        in_specs=[
            pl.BlockSpec(
                (block_r, b),
                functools.partial(lambda j, k: (n_streams * k + j, 0), j))
            for j in range(n_streams)
        ],
        out_specs=[
            pl.BlockSpec((1, b), lambda k: (0, 0)),
            pl.BlockSpec((1, b), lambda k: (0, 0)),
        ],
        out_shape=[
            jax.ShapeDtypeStruct((1, b), jnp.float32),
            jax.ShapeDtypeStruct((1, b), jnp.float32),
        ],
        scratch_shapes=[
            pltpu.VMEM((1, b), jnp.float32),
            pltpu.VMEM((1, b), jnp.float32),
        ],
        compiler_params=pltpu.CompilerParams(
            dimension_semantics=("arbitrary",),
        ),
    )(*([xt] * n_streams))

    out = pl.pallas_call(
        _epilogue_kernel,
        in_specs=[
            pl.BlockSpec((1, b), lambda: (0, 0)),
            pl.BlockSpec((1, b), lambda: (0, 0)),
            pl.BlockSpec((1, b), lambda: (0, 0)),
        ],
        out_specs=pl.BlockSpec((1, 1), lambda: (0, 0)),
        out_shape=jax.ShapeDtypeStruct((1, 1), jnp.float32),
    )(tval.reshape(1, b), m_v, s_v)
    return out[0, 0]


# speedup vs baseline: 1.0064x; 1.0064x over previous
import dataclasses
import functools
import math

import jax
import jax.numpy as jnp
from jax import lax
from jax.experimental import pallas as pl
from jax.experimental.pallas import tpu as pltpu
from jax.experimental.pallas import tpu_sc as plsc

_S = 64.0
_M = 0.15
_A = _S * math.log2(math.e)


def _sc_gather(xt, target):
    c, b = xt.shape
    info = plsc.get_sparse_core_info()
    nw = info.num_cores * info.num_subcores
    b_per_w = b // nw
    n_groups = b_per_w // 16
    mesh = plsc.VectorSubcoreMesh(core_axis_name="c", subcore_axis_name="s")
    cp = pltpu.CompilerParams()
    if "needs_layout_passes" in pltpu.CompilerParams.__dataclass_fields__:
        cp = dataclasses.replace(cp, needs_layout_passes=False)

    @functools.partial(
        pl.kernel,
        mesh=mesh,
        compiler_params=cp,
        out_type=jax.ShapeDtypeStruct((b,), jnp.float32),
        scratch_types=[
            pltpu.VMEM((b_per_w,), jnp.int32),
            pltpu.VMEM((b_per_w, b), jnp.float32),
            pltpu.VMEM((b_per_w,), jnp.float32),
        ],
    )
    def gather_kernel(xt_hbm, tgt_hbm, out_hbm, idx_v, rows_v, val_v):
        wid = lax.axis_index("s") * info.num_cores + lax.axis_index("c")
        base = wid * b_per_w
        pltpu.sync_copy(tgt_hbm.at[pl.ds(base, b_per_w)], idx_v)
        pltpu.sync_copy(xt_hbm.at[idx_v], rows_v)
        lane16 = lax.iota(jnp.int32, 16)
        for g in range(n_groups):
            row_idx = lane16 + (g * 16)
            col_idx = lane16 + (base + g * 16)
            vals = plsc.load_gather(rows_v, [row_idx, col_idx])
            val_v[pl.ds(g * 16, 16)] = vals
        pltpu.sync_copy(val_v, out_hbm.at[pl.ds(base, b_per_w)])

    return gather_kernel(xt, target)


def _stream_kernel(*refs, c_total, block_r, n_streams):
    x_refs = refs[:n_streams]
    m_ref, s_ref, m_sc, s_sc = refs[n_streams:]
    k = pl.program_id(0)
    nk = pl.num_programs(0)

    @pl.when(k == 0)
    def _init():
        m_sc[...] = jnp.full_like(m_sc, -jnp.inf)
        s_sc[...] = jnp.zeros_like(s_sc)

    def process(x):
        m_old = m_sc[...]
        m_new = jnp.maximum(m_old, jnp.max(x, axis=0, keepdims=True))
        y = jnp.exp2(x * _A - m_new * _A)
        s_sc[...] = s_sc[...] * jnp.exp2(_A * (m_old - m_new)) + jnp.sum(
            y, axis=0, keepdims=True)
        m_sc[...] = m_new

    if c_total == n_streams * nk * block_r:
        for x_ref in x_refs:
            process(x_ref[...])
    else:
        @pl.when(k < nk - 1)
        def _steady():
            for x_ref in x_refs:
                process(x_ref[...])

        @pl.when(k == nk - 1)
        def _last():
            valid = c_total - n_streams * (nk - 1) * block_r
            for j, x_ref in enumerate(x_refs):
                x = x_ref[...]
                pad = (jax.lax.broadcasted_iota(jnp.int32, x.shape, 0)
                       >= valid - j * block_r)
                process(jnp.where(pad, -jnp.inf, x))

    @pl.when(k == nk - 1)
    def _emit():
        m_ref[...] = m_sc[...]
        s_ref[...] = s_sc[...]


def _epilogue_kernel(tval_ref, m_ref, s_ref, out_ref):
    m = _S * m_ref[...]
    tv = _S * tval_ref[...]
    tm = tv - _S * _M
    se = s_ref[...] - jnp.exp(tv - m) + jnp.exp(tm - m)
    logpt = tm - m - jnp.log(se)
    out_ref[...] = jnp.full_like(out_ref, -jnp.mean(logpt))


def kernel(cos_theta, target):
    b, c = cos_theta.shape
    xt = cos_theta.T
    tval = _sc_gather(xt, target.astype(jnp.int32))

    n_streams = 2
    block_r = min(2000, c)
    grid = pl.cdiv(c, n_streams * block_r)

    m_v, s_v = pl.pallas_call(
        functools.partial(_stream_kernel, c_total=c, block_r=block_r,
                          n_streams=n_streams),
        grid=(grid,),
        in_specs=[
            pl.BlockSpec(
                (block_r, b),
                functools.partial(lambda j, k: (n_streams * k + j, 0), j))
            for j in range(n_streams)
        ],
        out_specs=[
            pl.BlockSpec((1, b), lambda k: (0, 0)),
            pl.BlockSpec((1, b), lambda k: (0, 0)),
        ],
        out_shape=[
            jax.ShapeDtypeStruct((1, b), jnp.float32),
            jax.ShapeDtypeStruct((1, b), jnp.float32),
        ],
        scratch_shapes=[
            pltpu.VMEM((1, b), jnp.float32),
            pltpu.VMEM((1, b), jnp.float32),
        ],
        compiler_params=pltpu.CompilerParams(
            dimension_semantics=("arbitrary",),
        ),
    )(*([xt] * n_streams))

    out = pl.pallas_call(
        _epilogue_kernel,
        in_specs=[
            pl.BlockSpec((1, b), lambda: (0, 0)),
            pl.BlockSpec((1, b), lambda: (0, 0)),
            pl.BlockSpec((1, b), lambda: (0, 0)),
        ],
        out_specs=pl.BlockSpec((1, 1), lambda: (0, 0)),
        out_shape=jax.ShapeDtypeStruct((1, 1), jnp.float32),
    )(tval.reshape(1, b), m_v, s_v)
    return out[0, 0]


# speedup vs baseline: 1.0066x; 1.0003x over previous
import dataclasses
import functools
import math

import jax
import jax.numpy as jnp
from jax import lax
from jax.experimental import pallas as pl
from jax.experimental.pallas import tpu as pltpu
from jax.experimental.pallas import tpu_sc as plsc

_S = 64.0
_M = 0.15
_A = _S * math.log2(math.e)


def _sc_gather(xt, target):
    c, b = xt.shape
    info = plsc.get_sparse_core_info()
    nw = info.num_cores * info.num_subcores
    b_per_w = b // nw
    n_groups = b_per_w // 16
    mesh = plsc.VectorSubcoreMesh(core_axis_name="c", subcore_axis_name="s")
    cp = pltpu.CompilerParams()
    if "needs_layout_passes" in pltpu.CompilerParams.__dataclass_fields__:
        cp = dataclasses.replace(cp, needs_layout_passes=False)

    @functools.partial(
        pl.kernel,
        mesh=mesh,
        compiler_params=cp,
        out_type=jax.ShapeDtypeStruct((b,), jnp.float32),
        scratch_types=[
            pltpu.VMEM((b_per_w,), jnp.int32),
            pltpu.VMEM((b_per_w, b), jnp.float32),
            pltpu.VMEM((b_per_w,), jnp.float32),
        ],
    )
    def gather_kernel(xt_hbm, tgt_hbm, out_hbm, idx_v, rows_v, val_v):
        wid = lax.axis_index("s") * info.num_cores + lax.axis_index("c")
        base = wid * b_per_w
        pltpu.sync_copy(tgt_hbm.at[pl.ds(base, b_per_w)], idx_v)
        pltpu.sync_copy(xt_hbm.at[idx_v], rows_v)
        lane16 = lax.iota(jnp.int32, 16)
        for g in range(n_groups):
            row_idx = lane16 + (g * 16)
            col_idx = lane16 + (base + g * 16)
            vals = plsc.load_gather(rows_v, [row_idx, col_idx])
            val_v[pl.ds(g * 16, 16)] = vals
        pltpu.sync_copy(val_v, out_hbm.at[pl.ds(base, b_per_w)])

    return gather_kernel(xt, target)


def _stream_kernel(*refs, c_total, block_r, n_streams):
    x_refs = refs[:n_streams]
    m_ref, s_ref, m_sc, s_sc = refs[n_streams:]
    k = pl.program_id(0)
    nk = pl.num_programs(0)

    @pl.when(k == 0)
    def _init():
        m_sc[...] = jnp.full_like(m_sc, -jnp.inf)
        s_sc[...] = jnp.zeros_like(s_sc)

    def process(x):
        m_old = m_sc[...]
        m_new = jnp.maximum(m_old, jnp.max(x, axis=0, keepdims=True))
        y = jnp.exp2(x * _A - m_new * _A)
        s_sc[...] = s_sc[...] * jnp.exp2(_A * (m_old - m_new)) + jnp.sum(
            y, axis=0, keepdims=True)
        m_sc[...] = m_new

    if c_total == n_streams * nk * block_r:
        for x_ref in x_refs:
            process(x_ref[...])
    else:
        @pl.when(k < nk - 1)
        def _steady():
            for x_ref in x_refs:
                process(x_ref[...])

        @pl.when(k == nk - 1)
        def _last():
            valid = c_total - n_streams * (nk - 1) * block_r
            for j, x_ref in enumerate(x_refs):
                vj = valid - j * block_r
                if vj >= block_r:
                    process(x_ref[...])
                elif vj > 0:
                    x = x_ref[...]
                    pad = (jax.lax.broadcasted_iota(jnp.int32, x.shape, 0)
                           >= vj)
                    process(jnp.where(pad, -jnp.inf, x))

    @pl.when(k == nk - 1)
    def _emit():
        m_ref[...] = m_sc[...]
        s_ref[...] = s_sc[...]


def _epilogue_kernel(tval_ref, m_ref, s_ref, out_ref):
    m = _S * m_ref[...]
    tv = _S * tval_ref[...]
    tm = tv - _S * _M
    se = s_ref[...] - jnp.exp(tv - m) + jnp.exp(tm - m)
    logpt = tm - m - jnp.log(se)
    out_ref[...] = jnp.full_like(out_ref, -jnp.mean(logpt))


def kernel(cos_theta, target):
    b, c = cos_theta.shape
    xt = cos_theta.T
    tval = _sc_gather(xt, target.astype(jnp.int32))

    n_streams = 2
    block_r = min(2000, c)
    grid = pl.cdiv(c, n_streams * block_r)

    m_v, s_v = pl.pallas_call(
        functools.partial(_stream_kernel, c_total=c, block_r=block_r,
                          n_streams=n_streams),
        grid=(grid,),
        in_specs=[
            pl.BlockSpec(
                (block_r, b),
                functools.partial(lambda j, k: (n_streams * k + j, 0), j))
            for j in range(n_streams)
        ],
        out_specs=[
            pl.BlockSpec((1, b), lambda k: (0, 0)),
            pl.BlockSpec((1, b), lambda k: (0, 0)),
        ],
        out_shape=[
            jax.ShapeDtypeStruct((1, b), jnp.float32),
            jax.ShapeDtypeStruct((1, b), jnp.float32),
        ],
        scratch_shapes=[
            pltpu.VMEM((1, b), jnp.float32),
            pltpu.VMEM((1, b), jnp.float32),
        ],
        compiler_params=pltpu.CompilerParams(
            dimension_semantics=("arbitrary",),
        ),
    )(*([xt] * n_streams))

    out = pl.pallas_call(
        _epilogue_kernel,
        in_specs=[
            pl.BlockSpec((1, b), lambda: (0, 0)),
            pl.BlockSpec((1, b), lambda: (0, 0)),
            pl.BlockSpec((1, b), lambda: (0, 0)),
        ],
        out_specs=pl.BlockSpec((1, 1), lambda: (0, 0)),
        out_shape=jax.ShapeDtypeStruct((1, 1), jnp.float32),
    )(tval.reshape(1, b), m_v, s_v)
    return out[0, 0]
